# Initial kernel scaffold; baseline (speedup 1.0000x reference)
import jax
import jax.numpy as jnp
from jax.experimental import pallas as pl


def kernel(x, edge_index, edge_weight, batch, W1, b1, gamma1, beta1, W2, b2, gamma2, beta2, Wfc, bfc):
    raise NotImplementedError("write your pallas kernel here")


# speedup vs baseline: 6.3300x; 6.3300x over previous
import functools

import jax
import jax.numpy as jnp
from jax import lax
from jax.experimental import pallas as pl
from jax.experimental.pallas import tpu as pltpu
from jax.experimental.pallas import tpu_sc as plsc

NC, NS, L = 2, 16, 16
NW = NC * NS
BLK = 128
NKC = 8
CH = BLK * NKC
EPS = 1e-5
G = 1024


def _sc_segment_accumulate(table, src_blk, dst_blk, w_flat, n_out):
    nb = src_blk.shape[0]
    assert nb % NW == 0
    nbw = nb // NW
    assert nbw % NKC == 0
    nchunks = nbw // NKC
    n_acc = ((n_out + BLK - 1) // BLK) * BLK
    Z = n_acc // NS

    mesh = plsc.VectorSubcoreMesh(
        core_axis_name="c", subcore_axis_name="s", num_cores=NC, num_subcores=NS)

    @functools.partial(
        pl.kernel,
        out_type=jax.ShapeDtypeStruct((NC, n_acc, L), jnp.float32),
        mesh=mesh,
        scratch_types=[
            pltpu.VMEM_SHARED((n_acc, L), jnp.float32),
            pltpu.VMEM((NKC, BLK), jnp.int32),
            pltpu.VMEM((NKC, BLK), jnp.int32),
            pltpu.VMEM((CH,), jnp.float32),
            pltpu.VMEM((CH, L), jnp.float32),
            pltpu.SemaphoreType.DMA,
        ],
        compiler_params=pltpu.CompilerParams(use_tc_tiling_on_sc=False),
    )
    def k(table_h, src_h, dst_h, w_h, out_h, acc, idx_s, idx_d, w_v, rows,
          sem):
        c = lax.axis_index("c")
        s = lax.axis_index("s")

        def zb(i, carry):
            rows[i, :] = jnp.zeros((L,), jnp.float32)
            return carry
        lax.fori_loop(0, CH, zb, 0)
        zoff = 0
        while zoff < Z:
            zn = min(CH, Z - zoff)
            pltpu.sync_copy(rows.at[pl.ds(0, zn)],
                            acc.at[pl.ds(s * Z + zoff, zn)])
            zoff += zn
        plsc.subcore_barrier()

        base_blk = c * (nb // 2) + s * nbw

        def chunk(ci, carry):
            b0 = base_blk + ci * NKC
            pltpu.sync_copy(src_h.at[pl.ds(b0, NKC)], idx_s)
            pltpu.sync_copy(dst_h.at[pl.ds(b0, NKC)], idx_d)
            pltpu.sync_copy(w_h.at[pl.ds(b0 * BLK, CH)], w_v)
            descs = [
                pltpu.make_async_copy(
                    table_h.at[idx_s.at[jb]],
                    rows.at[pl.ds(jb * BLK, BLK)], sem)
                for jb in range(NKC)
            ]
            for d in descs:
                d.start()
            for d in descs:
                d.wait()

            def scale(q, inner):
                wv = w_v[pl.ds(q * L, L)]
                for t in range(L):
                    r = q * L + t
                    rows[r, :] = rows[r, :] * wv[t]
                return inner
            lax.fori_loop(0, CH // L, scale, 0)

            for jb in range(NKC):
                pltpu.sync_copy(rows.at[pl.ds(jb * BLK, BLK)],
                                acc.at[idx_d.at[jb]], add=True)
            return carry
        lax.fori_loop(0, nchunks, chunk, 0)

        plsc.subcore_barrier()
        pltpu.sync_copy(acc.at[pl.ds(s * Z, Z)], out_h.at[c, pl.ds(s * Z, Z)])

    return k(table, src_blk, dst_blk, w_flat)


def _tc_layer1(agg1p, W1p, b1, g1, be1, n):
    Bn = 5000
    NB = n // Bn

    def body(aggp, W1r, b1r, g1r, be1r, h0, h1o, h2o, h3o, s1, s2):
        p = pl.program_id(0)
        A = aggp[0] + aggp[1]
        Y = jnp.dot(A, W1r[...], preferred_element_type=jnp.float32) + b1r[...]

        @pl.when(p == 0)
        def _():
            @pl.when(pl.program_id(1) == 0)
            def _():
                s1[...] = jnp.zeros_like(s1)
                s2[...] = jnp.zeros_like(s2)
            s1[...] += jnp.sum(Y, axis=0, keepdims=True)
            s2[...] += jnp.sum(Y * Y, axis=0, keepdims=True)

        @pl.when(p == 1)
        def _():
            mu = s1[...] * (1.0 / n)
            var = s2[...] * (1.0 / n) - mu * mu
            sc = g1r[...] / jnp.sqrt(var + EPS)
            H = (Y - mu) * sc + be1r[...]
            H = jnp.where(H >= 0, H, 0.01 * H)
            h0[...] = H[:, 0:16]
            h1o[...] = H[:, 16:32]
            h2o[...] = H[:, 32:48]
            h3o[...] = H[:, 48:64]

    out16 = jax.ShapeDtypeStruct((n, L), jnp.float32)
    return pl.pallas_call(
        body,
        grid=(2, NB),
        in_specs=[
            pl.BlockSpec((2, Bn, L), lambda p, i: (0, i, 0)),
            pl.BlockSpec((L, 64), lambda p, i: (0, 0)),
            pl.BlockSpec((1, 64), lambda p, i: (0, 0)),
            pl.BlockSpec((1, 64), lambda p, i: (0, 0)),
            pl.BlockSpec((1, 64), lambda p, i: (0, 0)),
        ],
        out_specs=[pl.BlockSpec((Bn, L), lambda p, i: (i, 0))] * 4,
        out_shape=[out16] * 4,
        scratch_shapes=[pltpu.VMEM((1, 64), jnp.float32)] * 2,
    )(agg1p, W1p, b1, g1, be1)


def _tc_layer2(a0, a1, a2, a3, W2, b2, g2, be2, Wfa, ba, n):
    Bn = 5000
    NB = n // Bn

    def body(p0r, p1r, p2r, p3r, W2r, b2r, g2r, be2r, Wfar, bar, Pout, s1, s2):
        p = pl.program_id(0)
        A = jnp.concatenate(
            [p0r[0] + p0r[1], p1r[0] + p1r[1], p2r[0] + p2r[1], p3r[0] + p3r[1]],
            axis=1)
        Y = jnp.dot(A, W2r[...], preferred_element_type=jnp.float32) + b2r[...]

        @pl.when(p == 0)
        def _():
            @pl.when(pl.program_id(1) == 0)
            def _():
                s1[...] = jnp.zeros_like(s1)
                s2[...] = jnp.zeros_like(s2)
            s1[...] += jnp.sum(Y, axis=0, keepdims=True)
            s2[...] += jnp.sum(Y * Y, axis=0, keepdims=True)

        @pl.when(p == 1)
        def _():
            mu = s1[...] * (1.0 / n)
            var = s2[...] * (1.0 / n) - mu * mu
            sc = g2r[...] / jnp.sqrt(var + EPS)
            H = (Y - mu) * sc + be2r[...]
            H = jnp.where(H >= 0, H, 0.01 * H)
            Pout[...] = jnp.dot(H, Wfar[...],
                                preferred_element_type=jnp.float32) + bar[...]

    spec16 = pl.BlockSpec((2, Bn, L), lambda p, i: (0, i, 0))
    return pl.pallas_call(
        body,
        grid=(2, NB),
        in_specs=[
            spec16, spec16, spec16, spec16,
            pl.BlockSpec((64, 128), lambda p, i: (0, 0)),
            pl.BlockSpec((1, 128), lambda p, i: (0, 0)),
            pl.BlockSpec((1, 128), lambda p, i: (0, 0)),
            pl.BlockSpec((1, 128), lambda p, i: (0, 0)),
            pl.BlockSpec((128, L), lambda p, i: (0, 0)),
            pl.BlockSpec((1, L), lambda p, i: (0, 0)),
        ],
        out_specs=pl.BlockSpec((Bn, L), lambda p, i: (i, 0)),
        out_shape=jax.ShapeDtypeStruct((n, L), jnp.float32),
        scratch_shapes=[pltpu.VMEM((1, 128), jnp.float32)] * 2,
    )(a0, a1, a2, a3, W2, b2, g2, be2, Wfa, ba)


def _tc_finalize(poolp, bfc):
    def body(pr, bfcr, out):
        S = pr[0] + pr[1]
        cnt = jnp.maximum(S[:, 2:3], 1.0)
        out[...] = S[:, 0:2] / cnt + bfcr[...]

    return pl.pallas_call(
        body,
        grid=(1,),
        in_specs=[
            pl.BlockSpec((2, G, L), lambda i: (0, 0, 0)),
            pl.BlockSpec((1, 2), lambda i: (0, 0)),
        ],
        out_specs=pl.BlockSpec((G, 2), lambda i: (0, 0)),
        out_shape=jax.ShapeDtypeStruct((G, 2), jnp.float32),
    )(poolp, bfc)


def kernel(x, edge_index, edge_weight, batch, W1, b1, gamma1, beta1,
           W2, b2, gamma2, beta2, Wfc, bfc):
    n, fin = x.shape
    e = edge_index.shape[1]

    x_pad = jnp.pad(x, ((0, 0), (0, L - fin)))
    ep = ((e + NW * CH - 1) // (NW * CH)) * (NW * CH)
    src = jnp.pad(edge_index[0], (0, ep - e)).reshape(-1, BLK)
    dst = jnp.pad(edge_index[1], (0, ep - e)).reshape(-1, BLK)
    w = jnp.pad(edge_weight, (0, ep - e))

    W1p = jnp.pad(W1, ((0, L - fin), (0, 0)))
    b1r = b1.reshape(1, 64)
    g1r = gamma1.reshape(1, 64)
    be1r = beta1.reshape(1, 64)
    b2r = b2.reshape(1, 128)
    g2r = gamma2.reshape(1, 128)
    be2r = beta2.reshape(1, 128)
    Wfa = jnp.pad(Wfc, ((0, 0), (0, L - 2)))
    ba = jnp.zeros((1, L), jnp.float32).at[0, 2].set(1.0)
    bfcr = bfc.reshape(1, 2)

    np_ = ((n + NW * CH - 1) // (NW * CH)) * (NW * CH)
    isrc = jnp.pad(jnp.arange(n, dtype=jnp.int32), (0, np_ - n)).reshape(-1, BLK)
    bdst = jnp.pad(batch, (0, np_ - n)).reshape(-1, BLK)
    bw = jnp.pad(jnp.ones((n,), jnp.float32), (0, np_ - n))

    agg1p = _sc_segment_accumulate(x_pad, src, dst, w, n)
    h1s = _tc_layer1(agg1p, W1p, b1r, g1r, be1r, n)
    a2 = [_sc_segment_accumulate(h, src, dst, w, n) for h in h1s]
    P = _tc_layer2(a2[0], a2[1], a2[2], a2[3], W2, b2r, g2r, be2r, Wfa, ba, n)
    poolp = _sc_segment_accumulate(P, isrc, bdst, bw, G)
    return _tc_finalize(poolp, bfcr)


# speedup vs baseline: 8.9772x; 1.4182x over previous
import functools

import jax
import jax.numpy as jnp
from jax import lax
from jax.experimental import pallas as pl
from jax.experimental.pallas import tpu as pltpu
from jax.experimental.pallas import tpu_sc as plsc

NC, NS, L = 2, 16, 16
NW = NC * NS
BLK = 128
NKC = 4
CH = BLK * NKC
NCH = 7
SUP = NKC * NCH
EPS = 1e-5
G = 1024


def _sc_segment_accumulate(table, src_blk, dst_blk, w_flat, n_out):
    nb = src_blk.shape[0]
    assert nb % (NW * SUP) == 0
    nbw = nb // NW
    nsup = nbw // SUP
    n_acc = ((n_out + BLK - 1) // BLK) * BLK
    Z = n_acc // NS

    mesh = plsc.VectorSubcoreMesh(
        core_axis_name="c", subcore_axis_name="s", num_cores=NC, num_subcores=NS)

    @functools.partial(
        pl.kernel,
        out_type=jax.ShapeDtypeStruct((NC, n_acc, L), jnp.float32),
        mesh=mesh,
        scratch_types=[
            pltpu.VMEM_SHARED((n_acc, L), jnp.float32),
            pltpu.VMEM((SUP, BLK), jnp.int32),
            pltpu.VMEM((SUP, BLK), jnp.int32),
            pltpu.VMEM((SUP * BLK,), jnp.float32),
            pltpu.VMEM((CH, L), jnp.float32),
            pltpu.VMEM((CH, L), jnp.float32),
            pltpu.SemaphoreType.DMA,
            pltpu.SemaphoreType.DMA,
            pltpu.SemaphoreType.DMA,
            pltpu.SemaphoreType.DMA,
        ],
        compiler_params=pltpu.CompilerParams(use_tc_tiling_on_sc=False),
    )
    def k(table_h, src_h, dst_h, w_h, out_h, acc, idx_s, idx_d, w_v,
          rows_a, rows_b, sem_ga, sem_gb, sem_sa, sem_sb):
        c = lax.axis_index("c")
        s = lax.axis_index("s")
        bufs = [(rows_a, sem_ga, sem_sa), (rows_b, sem_gb, sem_sb)]

        def zb(i, carry):
            rows_a[i, :] = jnp.zeros((L,), jnp.float32)
            return carry
        lax.fori_loop(0, CH, zb, 0)
        zoff = 0
        while zoff < Z:
            zn = min(CH, Z - zoff)
            pltpu.sync_copy(rows_a.at[pl.ds(0, zn)],
                            acc.at[pl.ds(s * Z + zoff, zn)])
            zoff += zn
        plsc.subcore_barrier()

        base_blk = c * (nb // 2) + s * nbw

        def g_descs(ck, buf):
            rows, sem_g, _ = buf
            return [
                pltpu.make_async_copy(
                    table_h.at[idx_s.at[ck * NKC + j]],
                    rows.at[pl.ds(j * BLK, BLK)], sem_g)
                for j in range(NKC)
            ]

        def s_descs(ck, buf):
            rows, _, sem_s = buf
            return [
                pltpu.make_async_copy(
                    rows.at[pl.ds(j * BLK, BLK)],
                    acc.at[idx_d.at[ck * NKC + j]], sem_s)
                for j in range(NKC)
            ]

        def scale(ck, buf):
            rows = buf[0]

            def sb(q, inner):
                wv = w_v[pl.ds(ck * CH + q * L, L)]
                for t in range(L):
                    r = q * L + t
                    rows[r, :] = rows[r, :] * wv[t]
                return inner
            lax.fori_loop(0, CH // L, sb, 0)

        def superchunk(si, carry):
            b0 = base_blk + si * SUP
            pltpu.sync_copy(src_h.at[pl.ds(b0, SUP)], idx_s)
            pltpu.sync_copy(dst_h.at[pl.ds(b0, SUP)], idx_d)
            pltpu.sync_copy(w_h.at[pl.ds(b0 * BLK, SUP * BLK)], w_v)
            for d in g_descs(0, bufs[0]):
                d.start()
            for ck in range(NCH):
                cur = bufs[ck % 2]
                nxt = bufs[(ck + 1) % 2]
                for d in g_descs(ck, cur):
                    d.wait()
                if ck + 1 < NCH:
                    if ck >= 1:
                        for d in s_descs(ck - 1, nxt):
                            d.wait()
                    for d in g_descs(ck + 1, nxt):
                        d.start()
                scale(ck, cur)
                for d in s_descs(ck, cur):
                    d.start(add=True)
            for d in s_descs(NCH - 2, bufs[(NCH - 2) % 2]):
                d.wait()
            for d in s_descs(NCH - 1, bufs[(NCH - 1) % 2]):
                d.wait()
            return carry
        lax.fori_loop(0, nsup, superchunk, 0)

        plsc.subcore_barrier()
        pltpu.sync_copy(acc.at[pl.ds(s * Z, Z)], out_h.at[c, pl.ds(s * Z, Z)])

    return k(table, src_blk, dst_blk, w_flat)


def _tc_layer1(agg1p, W1p, b1, g1, be1, n):
    Bn = 5000
    NB = n // Bn

    def body(aggp, W1r, b1r, g1r, be1r, h0, h1o, h2o, h3o, s1, s2):
        p = pl.program_id(0)
        A = aggp[0] + aggp[1]
        Y = jnp.dot(A, W1r[...], preferred_element_type=jnp.float32) + b1r[...]

        @pl.when(p == 0)
        def _():
            @pl.when(pl.program_id(1) == 0)
            def _():
                s1[...] = jnp.zeros_like(s1)
                s2[...] = jnp.zeros_like(s2)
            s1[...] += jnp.sum(Y, axis=0, keepdims=True)
            s2[...] += jnp.sum(Y * Y, axis=0, keepdims=True)

        @pl.when(p == 1)
        def _():
            mu = s1[...] * (1.0 / n)
            var = s2[...] * (1.0 / n) - mu * mu
            sc = g1r[...] / jnp.sqrt(var + EPS)
            H = (Y - mu) * sc + be1r[...]
            H = jnp.where(H >= 0, H, 0.01 * H)
            h0[...] = H[:, 0:16]
            h1o[...] = H[:, 16:32]
            h2o[...] = H[:, 32:48]
            h3o[...] = H[:, 48:64]

    out16 = jax.ShapeDtypeStruct((n, L), jnp.float32)
    return pl.pallas_call(
        body,
        grid=(2, NB),
        in_specs=[
            pl.BlockSpec((2, Bn, L), lambda p, i: (0, i, 0)),
            pl.BlockSpec((L, 64), lambda p, i: (0, 0)),
            pl.BlockSpec((1, 64), lambda p, i: (0, 0)),
            pl.BlockSpec((1, 64), lambda p, i: (0, 0)),
            pl.BlockSpec((1, 64), lambda p, i: (0, 0)),
        ],
        out_specs=[pl.BlockSpec((Bn, L), lambda p, i: (i, 0))] * 4,
        out_shape=[out16] * 4,
        scratch_shapes=[pltpu.VMEM((1, 64), jnp.float32)] * 2,
    )(agg1p, W1p, b1, g1, be1)


def _tc_layer2(a0, a1, a2, a3, W2, b2, g2, be2, Wfa, ba, n):
    Bn = 5000
    NB = n // Bn

    def body(p0r, p1r, p2r, p3r, W2r, b2r, g2r, be2r, Wfar, bar, Pout, s1, s2):
        p = pl.program_id(0)
        A = jnp.concatenate(
            [p0r[0] + p0r[1], p1r[0] + p1r[1], p2r[0] + p2r[1], p3r[0] + p3r[1]],
            axis=1)
        Y = jnp.dot(A, W2r[...], preferred_element_type=jnp.float32) + b2r[...]

        @pl.when(p == 0)
        def _():
            @pl.when(pl.program_id(1) == 0)
            def _():
                s1[...] = jnp.zeros_like(s1)
                s2[...] = jnp.zeros_like(s2)
            s1[...] += jnp.sum(Y, axis=0, keepdims=True)
            s2[...] += jnp.sum(Y * Y, axis=0, keepdims=True)

        @pl.when(p == 1)
        def _():
            mu = s1[...] * (1.0 / n)
            var = s2[...] * (1.0 / n) - mu * mu
            sc = g2r[...] / jnp.sqrt(var + EPS)
            H = (Y - mu) * sc + be2r[...]
            H = jnp.where(H >= 0, H, 0.01 * H)
            Pout[...] = jnp.dot(H, Wfar[...],
                                preferred_element_type=jnp.float32) + bar[...]

    spec16 = pl.BlockSpec((2, Bn, L), lambda p, i: (0, i, 0))
    return pl.pallas_call(
        body,
        grid=(2, NB),
        in_specs=[
            spec16, spec16, spec16, spec16,
            pl.BlockSpec((64, 128), lambda p, i: (0, 0)),
            pl.BlockSpec((1, 128), lambda p, i: (0, 0)),
            pl.BlockSpec((1, 128), lambda p, i: (0, 0)),
            pl.BlockSpec((1, 128), lambda p, i: (0, 0)),
            pl.BlockSpec((128, L), lambda p, i: (0, 0)),
            pl.BlockSpec((1, L), lambda p, i: (0, 0)),
        ],
        out_specs=pl.BlockSpec((Bn, L), lambda p, i: (i, 0)),
        out_shape=jax.ShapeDtypeStruct((n, L), jnp.float32),
        scratch_shapes=[pltpu.VMEM((1, 128), jnp.float32)] * 2,
    )(a0, a1, a2, a3, W2, b2, g2, be2, Wfa, ba)


def _tc_finalize(poolp, bfc):
    def body(pr, bfcr, out):
        S = pr[0] + pr[1]
        cnt = jnp.maximum(S[:, 2:3], 1.0)
        out[...] = S[:, 0:2] / cnt + bfcr[...]

    return pl.pallas_call(
        body,
        grid=(1,),
        in_specs=[
            pl.BlockSpec((2, G, L), lambda i: (0, 0, 0)),
            pl.BlockSpec((1, 2), lambda i: (0, 0)),
        ],
        out_specs=pl.BlockSpec((G, 2), lambda i: (0, 0)),
        out_shape=jax.ShapeDtypeStruct((G, 2), jnp.float32),
    )(poolp, bfc)


def kernel(x, edge_index, edge_weight, batch, W1, b1, gamma1, beta1,
           W2, b2, gamma2, beta2, Wfc, bfc):
    n, fin = x.shape
    e = edge_index.shape[1]

    x_pad = jnp.pad(x, ((0, 0), (0, L - fin)))
    grain = NW * SUP * BLK
    ep = ((e + grain - 1) // grain) * grain
    epad = (jnp.arange(ep - e, dtype=jnp.int32) % jnp.int32(n))
    src = jnp.concatenate([edge_index[0], epad]).reshape(-1, BLK)
    dst = jnp.concatenate([edge_index[1], epad]).reshape(-1, BLK)
    w = jnp.pad(edge_weight, (0, ep - e))

    W1p = jnp.pad(W1, ((0, L - fin), (0, 0)))
    b1r = b1.reshape(1, 64)
    g1r = gamma1.reshape(1, 64)
    be1r = beta1.reshape(1, 64)
    b2r = b2.reshape(1, 128)
    g2r = gamma2.reshape(1, 128)
    be2r = beta2.reshape(1, 128)
    Wfa = jnp.pad(Wfc, ((0, 0), (0, L - 2)))
    ba = jnp.zeros((1, L), jnp.float32).at[0, 2].set(1.0)
    bfcr = bfc.reshape(1, 2)

    np_ = ((n + grain - 1) // grain) * grain
    ppad = jnp.arange(np_ - n, dtype=jnp.int32)
    isrc = jnp.concatenate(
        [jnp.arange(n, dtype=jnp.int32), ppad % jnp.int32(n)]).reshape(-1, BLK)
    bdst = jnp.concatenate([batch, ppad % jnp.int32(G)]).reshape(-1, BLK)
    bw = jnp.pad(jnp.ones((n,), jnp.float32), (0, np_ - n))

    agg1p = _sc_segment_accumulate(x_pad, src, dst, w, n)
    h1s = _tc_layer1(agg1p, W1p, b1r, g1r, be1r, n)
    a2 = [_sc_segment_accumulate(h, src, dst, w, n) for h in h1s]
    P = _tc_layer2(a2[0], a2[1], a2[2], a2[3], W2, b2r, g2r, be2r, Wfa, ba, n)
    poolp = _sc_segment_accumulate(P, isrc, bdst, bw, G)
    return _tc_finalize(poolp, bfcr)


# speedup vs baseline: 12.8422x; 1.4305x over previous
import functools

import jax
import jax.numpy as jnp
from jax import lax
from jax.experimental import pallas as pl
from jax.experimental.pallas import tpu as pltpu
from jax.experimental.pallas import tpu_sc as plsc

NC, NS, L = 2, 16, 16
NW = NC * NS
BLK = 128
NKC = 4
CH = BLK * NKC
NCH = 7
SUP = NKC * NCH
EPS = 1e-5
G = 1024


def _sc_segment_accumulate(table, src_blk, dst_blk, w_flat, n_out):
    nb = src_blk.shape[0]
    assert nb % (NW * SUP) == 0
    nbw = nb // NW
    nsup = nbw // SUP
    n_acc = ((n_out + BLK - 1) // BLK) * BLK
    Z = n_acc // NS

    mesh = plsc.VectorSubcoreMesh(
        core_axis_name="c", subcore_axis_name="s", num_cores=NC, num_subcores=NS)

    @functools.partial(
        pl.kernel,
        out_type=jax.ShapeDtypeStruct((NC, n_acc, L), jnp.float32),
        mesh=mesh,
        scratch_types=[
            pltpu.VMEM_SHARED((n_acc, L), jnp.float32),
            pltpu.VMEM((SUP, BLK), jnp.int32),
            pltpu.VMEM((SUP, BLK), jnp.int32),
            pltpu.VMEM((SUP * BLK,), jnp.float32),
            pltpu.VMEM((CH, L), jnp.float32),
            pltpu.VMEM((CH, L), jnp.float32),
            pltpu.SemaphoreType.DMA,
            pltpu.SemaphoreType.DMA,
            pltpu.SemaphoreType.DMA,
            pltpu.SemaphoreType.DMA,
        ],
        compiler_params=pltpu.CompilerParams(use_tc_tiling_on_sc=False),
    )
    def k(table_h, src_h, dst_h, w_h, out_h, acc, idx_s, idx_d, w_v,
          rows_a, rows_b, sem_ga, sem_gb, sem_sa, sem_sb):
        c = lax.axis_index("c")
        s = lax.axis_index("s")
        bufs = [(rows_a, sem_ga, sem_sa), (rows_b, sem_gb, sem_sb)]

        def zb(i, carry):
            rows_a[i, :] = jnp.zeros((L,), jnp.float32)
            return carry
        lax.fori_loop(0, CH, zb, 0)
        zoff = 0
        while zoff < Z:
            zn = min(CH, Z - zoff)
            pltpu.sync_copy(rows_a.at[pl.ds(0, zn)],
                            acc.at[pl.ds(s * Z + zoff, zn)])
            zoff += zn
        plsc.subcore_barrier()

        base_blk = c * (nb // 2) + s * nbw

        def g_descs(ck, buf):
            rows, sem_g, _ = buf
            return [
                pltpu.make_async_copy(
                    table_h.at[idx_s.at[ck * NKC + j]],
                    rows.at[pl.ds(j * BLK, BLK)], sem_g)
                for j in range(NKC)
            ]

        def s_descs(ck, buf):
            rows, _, sem_s = buf
            return [
                pltpu.make_async_copy(
                    rows.at[pl.ds(j * BLK, BLK)],
                    acc.at[idx_d.at[ck * NKC + j]], sem_s)
                for j in range(NKC)
            ]

        def scale(ck, buf):
            rows = buf[0]

            def sb(q, inner):
                wv = w_v[pl.ds(ck * CH + q * L, L)]
                for t in range(L):
                    r = q * L + t
                    rows[r, :] = rows[r, :] * wv[t]
                return inner
            lax.fori_loop(0, CH // L, sb, 0)

        def superchunk(si, carry):
            b0 = base_blk + si * SUP
            pltpu.sync_copy(src_h.at[pl.ds(b0, SUP)], idx_s)
            pltpu.sync_copy(dst_h.at[pl.ds(b0, SUP)], idx_d)
            pltpu.sync_copy(w_h.at[pl.ds(b0 * BLK, SUP * BLK)], w_v)
            for d in g_descs(0, bufs[0]):
                d.start()
            for ck in range(NCH):
                cur = bufs[ck % 2]
                nxt = bufs[(ck + 1) % 2]
                for d in g_descs(ck, cur):
                    d.wait()
                if ck + 1 < NCH:
                    if ck >= 1:
                        for d in s_descs(ck - 1, nxt):
                            d.wait()
                    for d in g_descs(ck + 1, nxt):
                        d.start()
                scale(ck, cur)
                for d in s_descs(ck, cur):
                    d.start(add=True)
            for d in s_descs(NCH - 2, bufs[(NCH - 2) % 2]):
                d.wait()
            for d in s_descs(NCH - 1, bufs[(NCH - 1) % 2]):
                d.wait()
            return carry
        lax.fori_loop(0, nsup, superchunk, 0)

        plsc.subcore_barrier()
        pltpu.sync_copy(acc.at[pl.ds(s * Z, Z)], out_h.at[c, pl.ds(s * Z, Z)])

    return k(table, src_blk, dst_blk, w_flat)


def _fold8(s, d):
    acc = s[:, 0:d]
    for k in range(1, 8):
        acc = acc + s[:, k * d:(k + 1) * d]
    return acc


def _tc_layer1(agg1p8, W8, b1, b8, g8, be8, n, n_acc):
    n8 = n_acc // 8
    NB = 23
    Bn8 = n8 // NB
    assert Bn8 * NB == n8 and Bn8 % 8 == 0
    npad = n_acc - n

    def body(aggp, W8r, b1r, b8r, g8r, be8r, h0, h1o, h2o, h3o, s1, s2):
        p = pl.program_id(0)
        A = aggp[0] + aggp[1]
        Y = jnp.dot(A, W8r[...], preferred_element_type=jnp.float32) + b8r[...]

        @pl.when(p == 0)
        def _():
            @pl.when(pl.program_id(1) == 0)
            def _():
                s1[...] = jnp.zeros_like(s1)
                s2[...] = jnp.zeros_like(s2)
            s1[...] += jnp.sum(Y, axis=0, keepdims=True)
            s2[...] += jnp.sum(Y * Y, axis=0, keepdims=True)

        @pl.when(p == 1)
        def _():
            s1f = _fold8(s1[...], 64) - npad * b1r[...]
            s2f = _fold8(s2[...], 64) - npad * b1r[...] * b1r[...]
            mu = s1f * (1.0 / n)
            var = s2f * (1.0 / n) - mu * mu
            sc = g8r[...] / jnp.sqrt(jnp.concatenate([var] * 8, axis=1) + EPS)
            mu8 = jnp.concatenate([mu] * 8, axis=1)
            H = (Y - mu8) * sc + be8r[...]
            H = jnp.where(H >= 0, H, 0.01 * H)
            for j, out in enumerate((h0, h1o, h2o, h3o)):
                out[...] = jnp.concatenate(
                    [H[:, t * 64 + 16 * j: t * 64 + 16 * j + 16]
                     for t in range(8)], axis=1)

    out_pk = jax.ShapeDtypeStruct((n8, 128), jnp.float32)
    return pl.pallas_call(
        body,
        grid=(2, NB),
        in_specs=[
            pl.BlockSpec((2, Bn8, 128), lambda p, i: (0, i, 0)),
            pl.BlockSpec((128, 512), lambda p, i: (0, 0)),
            pl.BlockSpec((1, 64), lambda p, i: (0, 0)),
            pl.BlockSpec((1, 512), lambda p, i: (0, 0)),
            pl.BlockSpec((1, 512), lambda p, i: (0, 0)),
            pl.BlockSpec((1, 512), lambda p, i: (0, 0)),
        ],
        out_specs=[pl.BlockSpec((Bn8, 128), lambda p, i: (i, 0))] * 4,
        out_shape=[out_pk] * 4,
        scratch_shapes=[pltpu.VMEM((1, 512), jnp.float32)] * 2,
    )(agg1p8, W8, b1, b8, g8, be8)


def _tc_layer2(a0, a1, a2, a3, W28, b2, b28, g28, be28, Wf8, ba8, n, n_acc):
    n8 = n_acc // 8
    NB = 23
    Bn8 = n8 // NB
    assert Bn8 * NB == n8 and Bn8 % 8 == 0
    npad = n_acc - n

    def body(p0r, p1r, p2r, p3r, W28r, b2r, b28r, g28r, be28r, Wf8r, ba8r,
             Pout, s1, s2):
        p = pl.program_id(0)
        Y = b28r[...]
        for j, pr in enumerate((p0r, p1r, p2r, p3r)):
            Y = Y + jnp.dot(pr[0] + pr[1], W28r[j],
                            preferred_element_type=jnp.float32)

        @pl.when(p == 0)
        def _():
            @pl.when(pl.program_id(1) == 0)
            def _():
                s1[...] = jnp.zeros_like(s1)
                s2[...] = jnp.zeros_like(s2)
            s1[...] += jnp.sum(Y, axis=0, keepdims=True)
            s2[...] += jnp.sum(Y * Y, axis=0, keepdims=True)

        @pl.when(p == 1)
        def _():
            s1f = _fold8(s1[...], 128) - npad * b2r[...]
            s2f = _fold8(s2[...], 128) - npad * b2r[...] * b2r[...]
            mu = s1f * (1.0 / n)
            var = s2f * (1.0 / n) - mu * mu
            sc = g28r[...] / jnp.sqrt(jnp.concatenate([var] * 8, axis=1) + EPS)
            mu8 = jnp.concatenate([mu] * 8, axis=1)
            H = (Y - mu8) * sc + be28r[...]
            H = jnp.where(H >= 0, H, 0.01 * H)
            Pout[...] = jnp.dot(H, Wf8r[...],
                                preferred_element_type=jnp.float32) + ba8r[...]

    spec_pk = pl.BlockSpec((2, Bn8, 128), lambda p, i: (0, i, 0))
    return pl.pallas_call(
        body,
        grid=(2, NB),
        in_specs=[
            spec_pk, spec_pk, spec_pk, spec_pk,
            pl.BlockSpec((4, 128, 1024), lambda p, i: (0, 0, 0)),
            pl.BlockSpec((1, 128), lambda p, i: (0, 0)),
            pl.BlockSpec((1, 1024), lambda p, i: (0, 0)),
            pl.BlockSpec((1, 1024), lambda p, i: (0, 0)),
            pl.BlockSpec((1, 1024), lambda p, i: (0, 0)),
            pl.BlockSpec((1024, 128), lambda p, i: (0, 0)),
            pl.BlockSpec((1, 128), lambda p, i: (0, 0)),
        ],
        out_specs=pl.BlockSpec((Bn8, 128), lambda p, i: (i, 0)),
        out_shape=jax.ShapeDtypeStruct((n8, 128), jnp.float32),
        scratch_shapes=[pltpu.VMEM((1, 1024), jnp.float32)] * 2,
    )(a0, a1, a2, a3, W28, b2, b28, g28, be28, Wf8, ba8)


def _tc_finalize(poolp, bfc):
    def body(pr, bfcr, out):
        S = pr[0] + pr[1]
        cnt = jnp.maximum(S[:, 2:3], 1.0)
        out[...] = S[:, 0:2] / cnt + bfcr[...]

    return pl.pallas_call(
        body,
        grid=(1,),
        in_specs=[
            pl.BlockSpec((2, G, L), lambda i: (0, 0, 0)),
            pl.BlockSpec((1, 2), lambda i: (0, 0)),
        ],
        out_specs=pl.BlockSpec((G, 2), lambda i: (0, 0)),
        out_shape=jax.ShapeDtypeStruct((G, 2), jnp.float32),
    )(poolp, bfc)


def kernel(x, edge_index, edge_weight, batch, W1, b1, gamma1, beta1,
           W2, b2, gamma2, beta2, Wfc, bfc):
    n, fin = x.shape
    e = edge_index.shape[1]

    x_pad = jnp.pad(x, ((0, 0), (0, L - fin)))
    grain = NW * SUP * BLK
    ep = ((e + grain - 1) // grain) * grain
    epad = (jnp.arange(ep - e, dtype=jnp.int32) % jnp.int32(n))
    src = jnp.concatenate([edge_index[0], epad]).reshape(-1, BLK)
    dst = jnp.concatenate([edge_index[1], epad]).reshape(-1, BLK)
    w = jnp.pad(edge_weight, (0, ep - e))

    n_acc = ((n + BLK - 1) // BLK) * BLK
    n8 = n_acc // 8
    eye8 = jnp.eye(8, dtype=jnp.float32)
    W1p = jnp.pad(W1, ((0, L - fin), (0, 0)))
    W8 = jnp.kron(eye8, W1p)
    b1r = b1.reshape(1, 64)
    b8 = jnp.tile(b1r, (1, 8))
    g8 = jnp.tile(gamma1.reshape(1, 64), (1, 8))
    be8 = jnp.tile(beta1.reshape(1, 64), (1, 8))
    W28 = jnp.stack([jnp.kron(eye8, W2[16 * j:16 * (j + 1), :])
                     for j in range(4)])
    b2r = b2.reshape(1, 128)
    b28 = jnp.tile(b2r, (1, 8))
    g28 = jnp.tile(gamma2.reshape(1, 128), (1, 8))
    be28 = jnp.tile(beta2.reshape(1, 128), (1, 8))
    Wfa = jnp.pad(Wfc, ((0, 0), (0, L - 2)))
    Wf8 = jnp.kron(eye8, Wfa)
    ba = jnp.zeros((1, L), jnp.float32).at[0, 2].set(1.0)
    ba8 = jnp.tile(ba, (1, 8))
    bfcr = bfc.reshape(1, 2)

    np_ = ((n + grain - 1) // grain) * grain
    ppad = jnp.arange(np_ - n, dtype=jnp.int32)
    isrc = jnp.concatenate(
        [jnp.arange(n, dtype=jnp.int32), ppad % jnp.int32(n)]).reshape(-1, BLK)
    bdst = jnp.concatenate([batch, ppad % jnp.int32(G)]).reshape(-1, BLK)
    bw = jnp.pad(jnp.ones((n,), jnp.float32), (0, np_ - n))

    agg1p = _sc_segment_accumulate(x_pad, src, dst, w, n)
    h1s = _tc_layer1(agg1p.reshape(NC, n8, 128), W8, b1r, b8, g8, be8, n, n_acc)
    a2 = [_sc_segment_accumulate(h.reshape(n_acc, L), src, dst, w, n)
          for h in h1s]
    a2p = [a.reshape(NC, n8, 128) for a in a2]
    P = _tc_layer2(a2p[0], a2p[1], a2p[2], a2p[3], W28, b2r, b28, g28, be28,
                   Wf8, ba8, n, n_acc)
    poolp = _sc_segment_accumulate(P.reshape(n_acc, L), isrc, bdst, bw, G)
    return _tc_finalize(poolp, bfcr)


# speedup vs baseline: 13.6764x; 1.0650x over previous
import functools

import jax
import jax.numpy as jnp
from jax import lax
from jax.experimental import pallas as pl
from jax.experimental.pallas import tpu as pltpu
from jax.experimental.pallas import tpu_sc as plsc

NC, NS, L = 2, 16, 16
NW = NC * NS
BLK = 128
NKC = 4
CH = BLK * NKC
NCH = 7
SUP = NKC * NCH
EPS = 1e-5
G = 1024


def _sc_segment_accumulate(table, src_blk, dst_blk, w_flat, n_out):
    nb = src_blk.shape[0]
    assert nb % (NW * SUP) == 0
    nbw = nb // NW
    nsup = nbw // SUP
    n_acc = ((n_out + BLK - 1) // BLK) * BLK
    Z = n_acc // NS

    mesh = plsc.VectorSubcoreMesh(
        core_axis_name="c", subcore_axis_name="s", num_cores=NC, num_subcores=NS)

    @functools.partial(
        pl.kernel,
        out_type=jax.ShapeDtypeStruct((NC, n_acc, L), jnp.float32),
        mesh=mesh,
        scratch_types=[
            pltpu.VMEM_SHARED((n_acc, L), jnp.float32),
            pltpu.VMEM((SUP, BLK), jnp.int32),
            pltpu.VMEM((SUP, BLK), jnp.int32),
            pltpu.VMEM((SUP * BLK,), jnp.float32),
            pltpu.VMEM((CH, L), jnp.float32),
            pltpu.VMEM((CH, L), jnp.float32),
            pltpu.SemaphoreType.DMA,
            pltpu.SemaphoreType.DMA,
            pltpu.SemaphoreType.DMA,
            pltpu.SemaphoreType.DMA,
        ],
        compiler_params=pltpu.CompilerParams(use_tc_tiling_on_sc=False),
    )
    def k(table_h, src_h, dst_h, w_h, out_h, acc, idx_s, idx_d, w_v,
          rows_a, rows_b, sem_ga, sem_gb, sem_sa, sem_sb):
        c = lax.axis_index("c")
        s = lax.axis_index("s")
        bufs = [(rows_a, sem_ga, sem_sa), (rows_b, sem_gb, sem_sb)]

        def zb(i, carry):
            rows_a[i, :] = jnp.zeros((L,), jnp.float32)
            return carry
        lax.fori_loop(0, CH, zb, 0)
        zdescs = []
        zoff = 0
        while zoff < Z:
            zn = min(CH, Z - zoff)
            zdescs.append(pltpu.make_async_copy(
                rows_a.at[pl.ds(0, zn)],
                acc.at[pl.ds(s * Z + zoff, zn)], sem_ga))
            zoff += zn
        for d in zdescs:
            d.start()
        for d in zdescs:
            d.wait()
        plsc.subcore_barrier()

        base_blk = c * (nb // 2) + s * nbw

        def g_descs(ck, buf):
            rows, sem_g, _ = buf
            return [
                pltpu.make_async_copy(
                    table_h.at[idx_s.at[ck * NKC + j]],
                    rows.at[pl.ds(j * BLK, BLK)], sem_g)
                for j in range(NKC)
            ]

        def s_descs(ck, buf):
            rows, _, sem_s = buf
            return [
                pltpu.make_async_copy(
                    rows.at[pl.ds(j * BLK, BLK)],
                    acc.at[idx_d.at[ck * NKC + j]], sem_s)
                for j in range(NKC)
            ]

        def scale(ck, buf):
            rows = buf[0]

            def sb(q, inner):
                wv = w_v[pl.ds(ck * CH + q * L, L)]
                for t in range(L):
                    r = q * L + t
                    rows[r, :] = rows[r, :] * wv[t]
                return inner
            lax.fori_loop(0, CH // L, sb, 0)

        def superchunk(si, carry):
            b0 = base_blk + si * SUP
            edescs = [
                pltpu.make_async_copy(src_h.at[pl.ds(b0, SUP)], idx_s, sem_ga),
                pltpu.make_async_copy(dst_h.at[pl.ds(b0, SUP)], idx_d, sem_ga),
                pltpu.make_async_copy(
                    w_h.at[pl.ds(b0 * BLK, SUP * BLK)], w_v, sem_ga),
            ]
            for d in edescs:
                d.start()
            for d in edescs:
                d.wait()
            for d in g_descs(0, bufs[0]):
                d.start()
            for ck in range(NCH):
                cur = bufs[ck % 2]
                nxt = bufs[(ck + 1) % 2]
                for d in g_descs(ck, cur):
                    d.wait()
                if ck + 1 < NCH:
                    if ck >= 1:
                        for d in s_descs(ck - 1, nxt):
                            d.wait()
                    for d in g_descs(ck + 1, nxt):
                        d.start()
                scale(ck, cur)
                for d in s_descs(ck, cur):
                    d.start(add=True)
            for d in s_descs(NCH - 2, bufs[(NCH - 2) % 2]):
                d.wait()
            for d in s_descs(NCH - 1, bufs[(NCH - 1) % 2]):
                d.wait()
            return carry
        lax.fori_loop(0, nsup, superchunk, 0)

        plsc.subcore_barrier()
        pltpu.sync_copy(acc.at[pl.ds(s * Z, Z)], out_h.at[c, pl.ds(s * Z, Z)])

    return k(table, src_blk, dst_blk, w_flat)


def _fold8(s, d):
    acc = s[:, 0:d]
    for k in range(1, 8):
        acc = acc + s[:, k * d:(k + 1) * d]
    return acc


def _tc_layer1(agg1p8, W8, b1, b8, g8, be8, n, n_acc):
    n8 = n_acc // 8
    NB = 23
    Bn8 = n8 // NB
    assert Bn8 * NB == n8 and Bn8 % 8 == 0
    npad = n_acc - n

    def body(aggp, W8r, b1r, b8r, g8r, be8r, h0, h1o, h2o, h3o, s1, s2):
        p = pl.program_id(0)
        A = aggp[0] + aggp[1]
        Y = jnp.dot(A, W8r[...], preferred_element_type=jnp.float32) + b8r[...]

        @pl.when(p == 0)
        def _():
            @pl.when(pl.program_id(1) == 0)
            def _():
                s1[...] = jnp.zeros_like(s1)
                s2[...] = jnp.zeros_like(s2)
            s1[...] += jnp.sum(Y, axis=0, keepdims=True)
            s2[...] += jnp.sum(Y * Y, axis=0, keepdims=True)

        @pl.when(p == 1)
        def _():
            s1f = _fold8(s1[...], 64) - npad * b1r[...]
            s2f = _fold8(s2[...], 64) - npad * b1r[...] * b1r[...]
            mu = s1f * (1.0 / n)
            var = s2f * (1.0 / n) - mu * mu
            sc = g8r[...] / jnp.sqrt(jnp.concatenate([var] * 8, axis=1) + EPS)
            mu8 = jnp.concatenate([mu] * 8, axis=1)
            H = (Y - mu8) * sc + be8r[...]
            H = jnp.where(H >= 0, H, 0.01 * H)
            for j, out in enumerate((h0, h1o, h2o, h3o)):
                out[...] = jnp.concatenate(
                    [H[:, t * 64 + 16 * j: t * 64 + 16 * j + 16]
                     for t in range(8)], axis=1)

    out_pk = jax.ShapeDtypeStruct((n8, 128), jnp.float32)
    return pl.pallas_call(
        body,
        grid=(2, NB),
        in_specs=[
            pl.BlockSpec((2, Bn8, 128), lambda p, i: (0, i, 0)),
            pl.BlockSpec((128, 512), lambda p, i: (0, 0)),
            pl.BlockSpec((1, 64), lambda p, i: (0, 0)),
            pl.BlockSpec((1, 512), lambda p, i: (0, 0)),
            pl.BlockSpec((1, 512), lambda p, i: (0, 0)),
            pl.BlockSpec((1, 512), lambda p, i: (0, 0)),
        ],
        out_specs=[pl.BlockSpec((Bn8, 128), lambda p, i: (i, 0))] * 4,
        out_shape=[out_pk] * 4,
        scratch_shapes=[pltpu.VMEM((1, 512), jnp.float32)] * 2,
    )(agg1p8, W8, b1, b8, g8, be8)


def _tc_layer2(a0, a1, a2, a3, W28, b2, b28, g28, be28, Wf8, ba8, n, n_acc):
    n8 = n_acc // 8
    NB = 23
    Bn8 = n8 // NB
    assert Bn8 * NB == n8 and Bn8 % 8 == 0
    npad = n_acc - n

    def body(p0r, p1r, p2r, p3r, W28r, b2r, b28r, g28r, be28r, Wf8r, ba8r,
             Pout, s1, s2):
        p = pl.program_id(0)
        Y = b28r[...]
        for j, pr in enumerate((p0r, p1r, p2r, p3r)):
            Y = Y + jnp.dot(pr[0] + pr[1], W28r[j],
                            preferred_element_type=jnp.float32)

        @pl.when(p == 0)
        def _():
            @pl.when(pl.program_id(1) == 0)
            def _():
                s1[...] = jnp.zeros_like(s1)
                s2[...] = jnp.zeros_like(s2)
            s1[...] += jnp.sum(Y, axis=0, keepdims=True)
            s2[...] += jnp.sum(Y * Y, axis=0, keepdims=True)

        @pl.when(p == 1)
        def _():
            s1f = _fold8(s1[...], 128) - npad * b2r[...]
            s2f = _fold8(s2[...], 128) - npad * b2r[...] * b2r[...]
            mu = s1f * (1.0 / n)
            var = s2f * (1.0 / n) - mu * mu
            sc = g28r[...] / jnp.sqrt(jnp.concatenate([var] * 8, axis=1) + EPS)
            mu8 = jnp.concatenate([mu] * 8, axis=1)
            H = (Y - mu8) * sc + be28r[...]
            H = jnp.where(H >= 0, H, 0.01 * H)
            Pout[...] = jnp.dot(H, Wf8r[...],
                                preferred_element_type=jnp.float32) + ba8r[...]

    spec_pk = pl.BlockSpec((2, Bn8, 128), lambda p, i: (0, i, 0))
    return pl.pallas_call(
        body,
        grid=(2, NB),
        in_specs=[
            spec_pk, spec_pk, spec_pk, spec_pk,
            pl.BlockSpec((4, 128, 1024), lambda p, i: (0, 0, 0)),
            pl.BlockSpec((1, 128), lambda p, i: (0, 0)),
            pl.BlockSpec((1, 1024), lambda p, i: (0, 0)),
            pl.BlockSpec((1, 1024), lambda p, i: (0, 0)),
            pl.BlockSpec((1, 1024), lambda p, i: (0, 0)),
            pl.BlockSpec((1024, 128), lambda p, i: (0, 0)),
            pl.BlockSpec((1, 128), lambda p, i: (0, 0)),
        ],
        out_specs=pl.BlockSpec((Bn8, 128), lambda p, i: (i, 0)),
        out_shape=jax.ShapeDtypeStruct((n8, 128), jnp.float32),
        scratch_shapes=[pltpu.VMEM((1, 1024), jnp.float32)] * 2,
    )(a0, a1, a2, a3, W28, b2, b28, g28, be28, Wf8, ba8)


def _tc_finalize(poolp, bfc):
    def body(pr, bfcr, out):
        S = pr[0] + pr[1]
        cnt = jnp.maximum(S[:, 2:3], 1.0)
        out[...] = S[:, 0:2] / cnt + bfcr[...]

    return pl.pallas_call(
        body,
        grid=(1,),
        in_specs=[
            pl.BlockSpec((2, G, L), lambda i: (0, 0, 0)),
            pl.BlockSpec((1, 2), lambda i: (0, 0)),
        ],
        out_specs=pl.BlockSpec((G, 2), lambda i: (0, 0)),
        out_shape=jax.ShapeDtypeStruct((G, 2), jnp.float32),
    )(poolp, bfc)


def kernel(x, edge_index, edge_weight, batch, W1, b1, gamma1, beta1,
           W2, b2, gamma2, beta2, Wfc, bfc):
    n, fin = x.shape
    e = edge_index.shape[1]

    x_pad = jnp.pad(x, ((0, 0), (0, L - fin)))
    grain = NW * SUP * BLK
    ep = ((e + grain - 1) // grain) * grain
    epad = (jnp.arange(ep - e, dtype=jnp.int32) % jnp.int32(n))
    src = jnp.concatenate([edge_index[0], epad]).reshape(-1, BLK)
    dst = jnp.concatenate([edge_index[1], epad]).reshape(-1, BLK)
    w = jnp.pad(edge_weight, (0, ep - e))

    n_acc = ((n + BLK - 1) // BLK) * BLK
    n8 = n_acc // 8
    eye8 = jnp.eye(8, dtype=jnp.float32)
    W1p = jnp.pad(W1, ((0, L - fin), (0, 0)))
    W8 = jnp.kron(eye8, W1p)
    b1r = b1.reshape(1, 64)
    b8 = jnp.tile(b1r, (1, 8))
    g8 = jnp.tile(gamma1.reshape(1, 64), (1, 8))
    be8 = jnp.tile(beta1.reshape(1, 64), (1, 8))
    W28 = jnp.stack([jnp.kron(eye8, W2[16 * j:16 * (j + 1), :])
                     for j in range(4)])
    b2r = b2.reshape(1, 128)
    b28 = jnp.tile(b2r, (1, 8))
    g28 = jnp.tile(gamma2.reshape(1, 128), (1, 8))
    be28 = jnp.tile(beta2.reshape(1, 128), (1, 8))
    Wfa = jnp.pad(Wfc, ((0, 0), (0, L - 2)))
    Wf8 = jnp.kron(eye8, Wfa)
    ba = jnp.zeros((1, L), jnp.float32).at[0, 2].set(1.0)
    ba8 = jnp.tile(ba, (1, 8))
    bfcr = bfc.reshape(1, 2)

    np_ = ((n + grain - 1) // grain) * grain
    ppad = jnp.arange(np_ - n, dtype=jnp.int32)
    isrc = jnp.concatenate(
        [jnp.arange(n, dtype=jnp.int32), ppad % jnp.int32(n)]).reshape(-1, BLK)
    bdst = jnp.concatenate([batch, ppad % jnp.int32(G)]).reshape(-1, BLK)
    bw = jnp.pad(jnp.ones((n,), jnp.float32), (0, np_ - n))

    agg1p = _sc_segment_accumulate(x_pad, src, dst, w, n)
    h1s = _tc_layer1(agg1p.reshape(NC, n8, 128), W8, b1r, b8, g8, be8, n, n_acc)
    a2 = [_sc_segment_accumulate(h.reshape(n_acc, L), src, dst, w, n)
          for h in h1s]
    a2p = [a.reshape(NC, n8, 128) for a in a2]
    P = _tc_layer2(a2p[0], a2p[1], a2p[2], a2p[3], W28, b2r, b28, g28, be28,
                   Wf8, ba8, n, n_acc)
    poolp = _sc_segment_accumulate(P.reshape(n_acc, L), isrc, bdst, bw, G)
    return _tc_finalize(poolp, bfcr)


# speedup vs baseline: 18.1948x; 1.3304x over previous
import functools

import jax
import jax.numpy as jnp
from jax import lax
from jax.experimental import pallas as pl
from jax.experimental.pallas import tpu as pltpu
from jax.experimental.pallas import tpu_sc as plsc

NC, NS, L = 2, 16, 16
NW = NC * NS
BLK = 128
NKC = 4
CH = BLK * NKC
NCH = 7
SUP = NKC * NCH
EPS = 1e-5
G = 1024


def _sc_segment_accumulate(table, src_blk, dst_blk, w_flat, n_out):
    nb = src_blk.shape[0]
    assert nb % (NW * SUP) == 0
    nbw = nb // NW
    nsup = nbw // SUP
    n_acc = ((n_out + BLK - 1) // BLK) * BLK
    Z = n_acc // NS
    wid = table.shape[1]
    dt = table.dtype

    mesh = plsc.VectorSubcoreMesh(
        core_axis_name="c", subcore_axis_name="s", num_cores=NC, num_subcores=NS)

    @functools.partial(
        pl.kernel,
        out_type=jax.ShapeDtypeStruct((NC, n_acc, wid), dt),
        mesh=mesh,
        scratch_types=[
            pltpu.VMEM_SHARED((n_acc, wid), dt),
            pltpu.VMEM((SUP, BLK), jnp.int32),
            pltpu.VMEM((SUP, BLK), jnp.int32),
            pltpu.VMEM((SUP * BLK,), jnp.float32),
            pltpu.VMEM((CH, wid), dt),
            pltpu.VMEM((CH, wid), dt),
            pltpu.SemaphoreType.DMA,
            pltpu.SemaphoreType.DMA,
            pltpu.SemaphoreType.DMA,
            pltpu.SemaphoreType.DMA,
        ],
        compiler_params=pltpu.CompilerParams(
            use_tc_tiling_on_sc=False, needs_layout_passes=False),
    )
    def k(table_h, src_h, dst_h, w_h, out_h, acc, idx_s, idx_d, w_v,
          rows_a, rows_b, sem_ga, sem_gb, sem_sa, sem_sb):
        c = lax.axis_index("c")
        s = lax.axis_index("s")
        bufs = [(rows_a, sem_ga, sem_sa), (rows_b, sem_gb, sem_sb)]

        def zb(i, carry):
            rows_a[i, :] = jnp.zeros((wid,), dt)
            return carry
        lax.fori_loop(0, CH, zb, 0)
        zdescs = []
        zoff = 0
        while zoff < Z:
            zn = min(CH, Z - zoff)
            zdescs.append(pltpu.make_async_copy(
                rows_a.at[pl.ds(0, zn)],
                acc.at[pl.ds(s * Z + zoff, zn)], sem_ga))
            zoff += zn
        for d in zdescs:
            d.start()
        for d in zdescs:
            d.wait()
        plsc.subcore_barrier()

        base_blk = c * (nb // 2) + s * nbw

        def g_descs(ck, buf):
            rows, sem_g, _ = buf
            return [
                pltpu.make_async_copy(
                    table_h.at[idx_s.at[ck * NKC + j]],
                    rows.at[pl.ds(j * BLK, BLK)], sem_g)
                for j in range(NKC)
            ]

        def s_descs(ck, buf):
            rows, _, sem_s = buf
            return [
                pltpu.make_async_copy(
                    rows.at[pl.ds(j * BLK, BLK)],
                    acc.at[idx_d.at[ck * NKC + j]], sem_s)
                for j in range(NKC)
            ]

        def scale(ck, buf):
            rows = buf[0]

            def sb(q, inner):
                wv = w_v[pl.ds(ck * CH + q * L, L)]
                for t in range(L):
                    r = q * L + t
                    if dt == jnp.bfloat16:
                        lo, hi = plsc.unpack(
                            rows[r, :], format=plsc.PackFormat.INTERLEAVED)
                        rows[r, :] = plsc.pack(
                            lo * wv[t], hi * wv[t],
                            format=plsc.PackFormat.INTERLEAVED)
                    else:
                        rows[r, :] = rows[r, :] * wv[t]
                return inner
            lax.fori_loop(0, CH // L, sb, 0)

        def superchunk(si, carry):
            b0 = base_blk + si * SUP
            edescs = [
                pltpu.make_async_copy(src_h.at[pl.ds(b0, SUP)], idx_s, sem_ga),
                pltpu.make_async_copy(dst_h.at[pl.ds(b0, SUP)], idx_d, sem_ga),
                pltpu.make_async_copy(
                    w_h.at[pl.ds(b0 * BLK, SUP * BLK)], w_v, sem_ga),
            ]
            for d in edescs:
                d.start()
            for d in edescs:
                d.wait()
            for d in g_descs(0, bufs[0]):
                d.start()
            for ck in range(NCH):
                cur = bufs[ck % 2]
                nxt = bufs[(ck + 1) % 2]
                for d in g_descs(ck, cur):
                    d.wait()
                if ck + 1 < NCH:
                    if ck >= 1:
                        for d in s_descs(ck - 1, nxt):
                            d.wait()
                    for d in g_descs(ck + 1, nxt):
                        d.start()
                scale(ck, cur)
                for d in s_descs(ck, cur):
                    d.start(add=True)
            for d in s_descs(NCH - 2, bufs[(NCH - 2) % 2]):
                d.wait()
            for d in s_descs(NCH - 1, bufs[(NCH - 1) % 2]):
                d.wait()
            return carry
        lax.fori_loop(0, nsup, superchunk, 0)

        plsc.subcore_barrier()
        pltpu.sync_copy(acc.at[pl.ds(s * Z, Z)], out_h.at[c, pl.ds(s * Z, Z)])

    return k(table, src_blk, dst_blk, w_flat)


def _fold8(s, d):
    acc = s[:, 0:d]
    for k in range(1, 8):
        acc = acc + s[:, k * d:(k + 1) * d]
    return acc


def _tc_layer1(agg1p8, W8, b1, b8, g8, be8, n, n_acc):
    n8 = n_acc // 8
    NB = 23
    Bn8 = n8 // NB
    assert Bn8 * NB == n8 and Bn8 % 8 == 0
    npad = n_acc - n

    def body(aggp, W8r, b1r, b8r, g8r, be8r, h0, h1o, s1, s2):
        p = pl.program_id(0)
        A = aggp[0] + aggp[1]
        Y = jnp.dot(A, W8r[...], preferred_element_type=jnp.float32) + b8r[...]

        @pl.when(p == 0)
        def _():
            @pl.when(pl.program_id(1) == 0)
            def _():
                s1[...] = jnp.zeros_like(s1)
                s2[...] = jnp.zeros_like(s2)
            s1[...] += jnp.sum(Y, axis=0, keepdims=True)
            s2[...] += jnp.sum(Y * Y, axis=0, keepdims=True)

        @pl.when(p == 1)
        def _():
            s1f = _fold8(s1[...], 64) - npad * b1r[...]
            s2f = _fold8(s2[...], 64) - npad * b1r[...] * b1r[...]
            mu = s1f * (1.0 / n)
            var = s2f * (1.0 / n) - mu * mu
            sc = g8r[...] / jnp.sqrt(jnp.concatenate([var] * 8, axis=1) + EPS)
            mu8 = jnp.concatenate([mu] * 8, axis=1)
            H = (Y - mu8) * sc + be8r[...]
            H = jnp.where(H >= 0, H, 0.01 * H)
            for j, out in enumerate((h0, h1o)):
                out[...] = jnp.concatenate(
                    [H[:, t * 64 + 32 * j: t * 64 + 32 * j + 32]
                     for t in range(8)], axis=1).astype(jnp.bfloat16)

    out_pk = jax.ShapeDtypeStruct((n8, 256), jnp.bfloat16)
    return pl.pallas_call(
        body,
        grid=(2, NB),
        in_specs=[
            pl.BlockSpec((2, Bn8, 128), lambda p, i: (0, i, 0)),
            pl.BlockSpec((128, 512), lambda p, i: (0, 0)),
            pl.BlockSpec((1, 64), lambda p, i: (0, 0)),
            pl.BlockSpec((1, 512), lambda p, i: (0, 0)),
            pl.BlockSpec((1, 512), lambda p, i: (0, 0)),
            pl.BlockSpec((1, 512), lambda p, i: (0, 0)),
        ],
        out_specs=[pl.BlockSpec((Bn8, 256), lambda p, i: (i, 0))] * 2,
        out_shape=[out_pk] * 2,
        scratch_shapes=[pltpu.VMEM((1, 512), jnp.float32)] * 2,
    )(agg1p8, W8, b1, b8, g8, be8)


def _tc_layer2(a0, a1, W28, b2, b28, g28, be28, Wf8, ba8, n, n_acc):
    n8 = n_acc // 8
    NB = 23
    Bn8 = n8 // NB
    assert Bn8 * NB == n8 and Bn8 % 8 == 0
    npad = n_acc - n

    def body(p0r, p1r, W28r, b2r, b28r, g28r, be28r, Wf8r, ba8r,
             Pout, s1, s2):
        p = pl.program_id(0)
        Y = b28r[...]
        for j, pr in enumerate((p0r, p1r)):
            A = pr[0].astype(jnp.float32) + pr[1].astype(jnp.float32)
            Y = Y + jnp.dot(A, W28r[j], preferred_element_type=jnp.float32)

        @pl.when(p == 0)
        def _():
            @pl.when(pl.program_id(1) == 0)
            def _():
                s1[...] = jnp.zeros_like(s1)
                s2[...] = jnp.zeros_like(s2)
            s1[...] += jnp.sum(Y, axis=0, keepdims=True)
            s2[...] += jnp.sum(Y * Y, axis=0, keepdims=True)

        @pl.when(p == 1)
        def _():
            s1f = _fold8(s1[...], 128) - npad * b2r[...]
            s2f = _fold8(s2[...], 128) - npad * b2r[...] * b2r[...]
            mu = s1f * (1.0 / n)
            var = s2f * (1.0 / n) - mu * mu
            sc = g28r[...] / jnp.sqrt(jnp.concatenate([var] * 8, axis=1) + EPS)
            mu8 = jnp.concatenate([mu] * 8, axis=1)
            H = (Y - mu8) * sc + be28r[...]
            H = jnp.where(H >= 0, H, 0.01 * H)
            Pout[...] = jnp.dot(H, Wf8r[...],
                                preferred_element_type=jnp.float32) + ba8r[...]

    spec_pk = pl.BlockSpec((2, Bn8, 256), lambda p, i: (0, i, 0))
    return pl.pallas_call(
        body,
        grid=(2, NB),
        in_specs=[
            spec_pk, spec_pk,
            pl.BlockSpec((2, 256, 1024), lambda p, i: (0, 0, 0)),
            pl.BlockSpec((1, 128), lambda p, i: (0, 0)),
            pl.BlockSpec((1, 1024), lambda p, i: (0, 0)),
            pl.BlockSpec((1, 1024), lambda p, i: (0, 0)),
            pl.BlockSpec((1, 1024), lambda p, i: (0, 0)),
            pl.BlockSpec((1024, 128), lambda p, i: (0, 0)),
            pl.BlockSpec((1, 128), lambda p, i: (0, 0)),
        ],
        out_specs=pl.BlockSpec((Bn8, 128), lambda p, i: (i, 0)),
        out_shape=jax.ShapeDtypeStruct((n8, 128), jnp.float32),
        scratch_shapes=[pltpu.VMEM((1, 1024), jnp.float32)] * 2,
    )(a0, a1, W28, b2, b28, g28, be28, Wf8, ba8)


def _tc_finalize(poolp, bfc):
    def body(pr, bfcr, out):
        S = pr[0] + pr[1]
        cnt = jnp.maximum(S[:, 2:3], 1.0)
        out[...] = S[:, 0:2] / cnt + bfcr[...]

    return pl.pallas_call(
        body,
        grid=(1,),
        in_specs=[
            pl.BlockSpec((2, G, L), lambda i: (0, 0, 0)),
            pl.BlockSpec((1, 2), lambda i: (0, 0)),
        ],
        out_specs=pl.BlockSpec((G, 2), lambda i: (0, 0)),
        out_shape=jax.ShapeDtypeStruct((G, 2), jnp.float32),
    )(poolp, bfc)


def kernel(x, edge_index, edge_weight, batch, W1, b1, gamma1, beta1,
           W2, b2, gamma2, beta2, Wfc, bfc):
    n, fin = x.shape
    e = edge_index.shape[1]

    x_pad = jnp.pad(x, ((0, 0), (0, L - fin)))
    grain = NW * SUP * BLK
    ep = ((e + grain - 1) // grain) * grain
    epad = (jnp.arange(ep - e, dtype=jnp.int32) % jnp.int32(n))
    src = jnp.concatenate([edge_index[0], epad]).reshape(-1, BLK)
    dst = jnp.concatenate([edge_index[1], epad]).reshape(-1, BLK)
    w = jnp.pad(edge_weight, (0, ep - e))

    n_acc = ((n + BLK - 1) // BLK) * BLK
    n8 = n_acc // 8
    eye8 = jnp.eye(8, dtype=jnp.float32)
    W1p = jnp.pad(W1, ((0, L - fin), (0, 0)))
    W8 = jnp.kron(eye8, W1p)
    b1r = b1.reshape(1, 64)
    b8 = jnp.tile(b1r, (1, 8))
    g8 = jnp.tile(gamma1.reshape(1, 64), (1, 8))
    be8 = jnp.tile(beta1.reshape(1, 64), (1, 8))
    W28 = jnp.stack([jnp.kron(eye8, W2[32 * j:32 * (j + 1), :])
                     for j in range(2)])
    b2r = b2.reshape(1, 128)
    b28 = jnp.tile(b2r, (1, 8))
    g28 = jnp.tile(gamma2.reshape(1, 128), (1, 8))
    be28 = jnp.tile(beta2.reshape(1, 128), (1, 8))
    Wfa = jnp.pad(Wfc, ((0, 0), (0, L - 2)))
    Wf8 = jnp.kron(eye8, Wfa)
    ba = jnp.zeros((1, L), jnp.float32).at[0, 2].set(1.0)
    ba8 = jnp.tile(ba, (1, 8))
    bfcr = bfc.reshape(1, 2)

    np_ = ((n + grain - 1) // grain) * grain
    ppad = jnp.arange(np_ - n, dtype=jnp.int32)
    isrc = jnp.concatenate(
        [jnp.arange(n, dtype=jnp.int32), ppad % jnp.int32(n)]).reshape(-1, BLK)
    bdst = jnp.concatenate([batch, ppad % jnp.int32(G)]).reshape(-1, BLK)
    bw = jnp.pad(jnp.ones((n,), jnp.float32), (0, np_ - n))

    agg1p = _sc_segment_accumulate(x_pad, src, dst, w, n)
    h1s = _tc_layer1(agg1p.reshape(NC, n8, 128), W8, b1r, b8, g8, be8, n, n_acc)
    a2 = [_sc_segment_accumulate(h.reshape(n_acc, 32), src, dst, w, n)
          for h in h1s]
    a2p = [a.reshape(NC, n8, 256) for a in a2]
    P = _tc_layer2(a2p[0], a2p[1], W28, b2r, b28, g28, be28,
                   Wf8, ba8, n, n_acc)
    poolp = _sc_segment_accumulate(P.reshape(n_acc, L), isrc, bdst, bw, G)
    return _tc_finalize(poolp, bfcr)
